# baseline (speedup 1.0000x reference)
import functools

import jax
import jax.numpy as jnp
from jax import lax
from jax.experimental import pallas as pl
from jax.experimental.pallas import tpu as pltpu
from jax.experimental.pallas import tpu_sc as plsc

_LANES = 16


@functools.lru_cache(maxsize=None)
def _make_embed_kernel(n_tokens: int, d_model: int, vocab: int, max_pos: int):
    info = plsc.get_sparse_core_info()
    num_cores, num_subcores = info.num_cores, info.num_subcores
    nw = num_cores * num_subcores
    assert n_tokens % nw == 0
    per_w = n_tokens // nw
    ch = min(per_w, 32)
    assert per_w % ch == 0 and ch % 8 == 0
    n_chunks = per_w // ch
    groups_per_row = d_model // _LANES

    mesh = plsc.VectorSubcoreMesh(core_axis_name="c", subcore_axis_name="s")

    @functools.partial(
        pl.kernel,
        mesh=mesh,
        out_type=jax.ShapeDtypeStruct((n_tokens, d_model), jnp.float32),
        scratch_types=[
            pltpu.VMEM((ch,), jnp.int32),
            pltpu.VMEM((ch,), jnp.int32),
            pltpu.VMEM((ch, d_model), jnp.float32),
            pltpu.VMEM((ch, d_model), jnp.float32),
            pltpu.SemaphoreType.DMA,
            pltpu.SemaphoreType.DMA,
        ],
    )
    def embed(ids_hbm, pos_hbm, wte_hbm, wpe_hbm, out_hbm,
              idx_v, pidx_v, rows_v, pos_rows_v, sem_a, sem_b):
        wid = lax.axis_index("s") * num_cores + lax.axis_index("c")
        base = wid * per_w

        def chunk_body(c, carry):
            off = pl.multiple_of(base + c * ch, ch)
            pltpu.sync_copy(ids_hbm.at[pl.ds(off, ch)], idx_v)
            pltpu.sync_copy(pos_hbm.at[pl.ds(off, ch)], pidx_v)
            g_tok = pltpu.async_copy(wte_hbm.at[idx_v], rows_v, sem_a)
            g_pos = pltpu.async_copy(wpe_hbm.at[pidx_v], pos_rows_v, sem_b)
            g_tok.wait()
            g_pos.wait()

            def add_body(g, acc):
                r = lax.div(g, groups_per_row)
                col = lax.rem(g, groups_per_row) * _LANES
                rows_v[r, pl.ds(col, _LANES)] = (
                    rows_v[r, pl.ds(col, _LANES)]
                    + pos_rows_v[r, pl.ds(col, _LANES)]
                )
                return acc

            lax.fori_loop(0, ch * groups_per_row, add_body, 0)
            pltpu.sync_copy(rows_v, out_hbm.at[pl.ds(off, ch)])
            return carry

        lax.fori_loop(0, n_chunks, chunk_body, 0)

    return embed


def kernel(input_ids, wte, wpe, past_length=0):
    batch, seq = input_ids.shape
    vocab, d_model = wte.shape
    max_pos = wpe.shape[0]
    n_tokens = batch * seq

    ids = jnp.clip(input_ids.reshape(-1).astype(jnp.int32), 0, vocab - 1)
    pos = (jnp.arange(n_tokens, dtype=jnp.int32) % seq
           + jnp.asarray(past_length, jnp.int32))
    pos = jnp.clip(pos, 0, max_pos - 1)

    out = _make_embed_kernel(n_tokens, d_model, vocab, max_pos)(
        ids, pos, wte, wpe)
    return out.reshape(batch, seq, d_model)


# speedup vs baseline: 2.4934x; 2.4934x over previous
import functools

import jax
import jax.numpy as jnp
from jax import lax
from jax.experimental import pallas as pl
from jax.experimental.pallas import tpu as pltpu
from jax.experimental.pallas import tpu_sc as plsc

_LANES = 16


@functools.lru_cache(maxsize=None)
def _make_embed_kernel(batch: int, seq: int, d_model: int):
    info = plsc.get_sparse_core_info()
    num_cores, num_subcores = info.num_cores, info.num_subcores
    nw = num_cores * num_subcores
    n_tokens = batch * seq
    assert seq % nw == 0
    pos_per_w = seq // nw
    per_w = batch * pos_per_w
    ch = 16
    assert pos_per_w % ch == 0
    ch_per_blk = pos_per_w // ch
    n_chunks = batch * ch_per_blk
    groups = d_model // _LANES

    mesh = plsc.VectorSubcoreMesh(core_axis_name="c", subcore_axis_name="s")

    @functools.partial(
        pl.kernel,
        mesh=mesh,
        out_type=jax.ShapeDtypeStruct((n_tokens, d_model), jnp.float32),
        scratch_types=[
            pltpu.VMEM((per_w,), jnp.int32),
            pltpu.VMEM((pos_per_w,), jnp.int32),
            pltpu.VMEM((pos_per_w, d_model), jnp.float32),
            pltpu.VMEM((ch, d_model), jnp.float32),
            pltpu.VMEM((ch, d_model), jnp.float32),
            pltpu.SemaphoreType.DMA,
            pltpu.SemaphoreType.DMA,
            pltpu.SemaphoreType.DMA,
            pltpu.SemaphoreType.DMA,
            pltpu.SemaphoreType.DMA,
        ],
    )
    def embed(ids_hbm, pos_hbm, wte_hbm, wpe_hbm, out_hbm,
              idx_v, pidx_v, wpe_v, rows0_v, rows1_v,
              sem_w, sem_g0, sem_g1, sem_o0, sem_o1):
        wid = lax.axis_index("s") * num_cores + lax.axis_index("c")
        rows = (rows0_v, rows1_v)
        sem_g = (sem_g0, sem_g1)
        sem_o = (sem_o0, sem_o1)

        pltpu.sync_copy(ids_hbm.at[pl.ds(wid * per_w, per_w)], idx_v)
        pltpu.sync_copy(pos_hbm.at[pl.ds(wid * pos_per_w, pos_per_w)], pidx_v)
        wpe_d = pltpu.async_copy(wpe_hbm.at[pidx_v], wpe_v, sem_w)

        def start_gather(t):
            p = t % 2
            return pltpu.async_copy(
                wte_hbm.at[idx_v.at[pl.ds(t * ch, ch)]], rows[p], sem_g[p])

        gd = [None, None]
        od = [None, None]
        gd[0] = start_gather(0)
        wpe_d.wait()

        for t in range(n_chunks):
            p = t % 2
            if t + 1 < n_chunks:
                if t >= 1:
                    od[1 - p].wait()
                gd[1 - p] = start_gather(t + 1)
            gd[p].wait()

            wrow = (t % ch_per_blk) * ch
            buf = rows[p]

            @plsc.parallel_loop(0, ch * groups, step=1, unroll=8)
            def _add(g, buf=buf, wrow=wrow):
                r = lax.div(g, groups)
                col = lax.rem(g, groups) * _LANES
                buf[r, pl.ds(col, _LANES)] = (
                    buf[r, pl.ds(col, _LANES)]
                    + wpe_v[wrow + r, pl.ds(col, _LANES)]
                )

            b = t // ch_per_blk
            off = b * seq + wid * pos_per_w + (t % ch_per_blk) * ch
            od[p] = pltpu.async_copy(buf, out_hbm.at[pl.ds(off, ch)], sem_o[p])

        od[0].wait()
        od[1].wait()

    return embed


def kernel(input_ids, wte, wpe, past_length=0):
    batch, seq = input_ids.shape
    vocab, d_model = wte.shape
    max_pos = wpe.shape[0]
    n_tokens = batch * seq
    info = plsc.get_sparse_core_info()
    nw = info.num_cores * info.num_subcores
    pos_per_w = seq // nw

    ids = jnp.clip(input_ids.astype(jnp.int32), 0, vocab - 1)
    ids = ids.reshape(batch, nw, pos_per_w).transpose(1, 0, 2).reshape(-1)
    pos = (jnp.arange(seq, dtype=jnp.int32)
           + jnp.asarray(past_length, jnp.int32))
    pos = jnp.clip(pos, 0, max_pos - 1)

    out = _make_embed_kernel(batch, seq, d_model)(ids, pos, wte, wpe)
    return out.reshape(batch, seq, d_model)
